# baseline (device time: 6753 ns/iter reference)
import jax
import jax.numpy as jnp
from jax import lax
from jax.experimental import pallas as pl
from jax.experimental.pallas import tpu as pltpu

C = 2


def kernel(x):
    _, m, n = x.shape
    half = n // 2
    mc = m // C
    x = pltpu.with_memory_space_constraint(x, pltpu.MemorySpace.HBM)

    def body(
        x_hbm,
        out_hbm,
        peer_f32,
        mine_f32,
        send_buf,
        recv_buf,
        acc,
        in_sems,
        out_sems,
        send_sems,
        recv_sems,
    ):
        my_x = lax.axis_index("x")
        my_y = lax.axis_index("y")
        peer_y = 1 - my_y

        barrier_sem = pltpu.get_barrier_semaphore()
        pl.semaphore_signal(
            barrier_sem,
            inc=1,
            device_id=(my_x, peer_y),
            device_id_type=pl.DeviceIdType.MESH,
        )

        peer_dmas = []
        for c in range(C):
            d = pltpu.make_async_copy(
                x_hbm.at[0, pl.ds(c * mc, mc), pl.ds(peer_y * half, half)],
                peer_f32.at[c],
                in_sems.at[c],
            )
            d.start()
            peer_dmas.append(d)
        dma_mine = pltpu.make_async_copy(
            x_hbm.at[0, :, pl.ds(my_y * half, half)], mine_f32, in_sems.at[C]
        )
        dma_mine.start()

        rdmas = []
        for c in range(C):
            peer_dmas[c].wait()
            send_buf[c, :, :] = peer_f32[c, :, :].astype(jnp.bfloat16)
            if c == 0:
                pl.semaphore_wait(barrier_sem, 1)
            rdma = pltpu.make_async_remote_copy(
                src_ref=send_buf.at[c],
                dst_ref=recv_buf.at[c],
                send_sem=send_sems.at[c],
                recv_sem=recv_sems.at[c],
                device_id=(my_x, peer_y),
                device_id_type=pl.DeviceIdType.MESH,
            )
            rdma.start()
            rdmas.append(rdma)

        dma_mine.wait()

        out_dmas = []
        for c in range(C):
            rdmas[c].wait_recv()
            acc[c, :, :] = (
                mine_f32[pl.ds(c * mc, mc), :]
                + recv_buf[c, :, :].astype(jnp.float32)
            ).astype(jnp.bfloat16)
            d = pltpu.make_async_copy(
                acc.at[c], out_hbm.at[pl.ds(c * mc, mc), :], out_sems.at[c]
            )
            d.start()
            out_dmas.append(d)

        for c in range(C):
            out_dmas[c].wait()
            rdmas[c].wait_send()

    return pl.pallas_call(
        body,
        out_shape=jax.ShapeDtypeStruct((m, half), jnp.bfloat16),
        in_specs=[pl.BlockSpec(memory_space=pltpu.MemorySpace.HBM)],
        out_specs=pl.BlockSpec(memory_space=pltpu.MemorySpace.HBM),
        scratch_shapes=[
            pltpu.VMEM((C, mc, half), jnp.float32),
            pltpu.VMEM((m, half), jnp.float32),
            pltpu.VMEM((C, mc, half), jnp.bfloat16),
            pltpu.VMEM((C, mc, half), jnp.bfloat16),
            pltpu.VMEM((C, mc, half), jnp.bfloat16),
            pltpu.SemaphoreType.DMA((C + 1,)),
            pltpu.SemaphoreType.DMA((C,)),
            pltpu.SemaphoreType.DMA((C,)),
            pltpu.SemaphoreType.DMA((C,)),
        ],
        compiler_params=pltpu.CompilerParams(collective_id=0),
    )(x)


# device time: 6623 ns/iter; 1.0196x vs baseline; 1.0196x over previous
import jax
import jax.numpy as jnp
from jax import lax
from jax.experimental import pallas as pl
from jax.experimental.pallas import tpu as pltpu

C = 2


def kernel(x):
    _, m, n = x.shape
    half = n // 2
    mc = m // C
    x = pltpu.with_memory_space_constraint(x, pltpu.MemorySpace.HBM)

    def body(
        x_hbm,
        out_ref,
        peer_f32,
        mine_f32,
        send_buf,
        recv_buf,
        in_sems,
        send_sems,
        recv_sems,
    ):
        my_x = lax.axis_index("x")
        my_y = lax.axis_index("y")
        peer_y = 1 - my_y

        barrier_sem = pltpu.get_barrier_semaphore()
        pl.semaphore_signal(
            barrier_sem,
            inc=1,
            device_id=(my_x, peer_y),
            device_id_type=pl.DeviceIdType.MESH,
        )

        peer_dmas = []
        for c in range(C):
            d = pltpu.make_async_copy(
                x_hbm.at[0, pl.ds(c * mc, mc), pl.ds(peer_y * half, half)],
                peer_f32.at[c],
                in_sems.at[c],
            )
            d.start()
            peer_dmas.append(d)
        dma_mine = pltpu.make_async_copy(
            x_hbm.at[0, :, pl.ds(my_y * half, half)], mine_f32, in_sems.at[C]
        )
        dma_mine.start()

        rdmas = []
        for c in range(C):
            peer_dmas[c].wait()
            send_buf[c, :, :] = peer_f32[c, :, :].astype(jnp.bfloat16)
            if c == 0:
                pl.semaphore_wait(barrier_sem, 1)
            rdma = pltpu.make_async_remote_copy(
                src_ref=send_buf.at[c],
                dst_ref=recv_buf.at[c],
                send_sem=send_sems.at[c],
                recv_sem=recv_sems.at[c],
                device_id=(my_x, peer_y),
                device_id_type=pl.DeviceIdType.MESH,
            )
            rdma.start()
            rdmas.append(rdma)

        dma_mine.wait()

        for c in range(C):
            rdmas[c].wait_recv()
            out_ref[pl.ds(c * mc, mc), :] = (
                mine_f32[pl.ds(c * mc, mc), :]
                + recv_buf[c, :, :].astype(jnp.float32)
            ).astype(jnp.bfloat16)

        for c in range(C):
            rdmas[c].wait_send()

    return pl.pallas_call(
        body,
        out_shape=jax.ShapeDtypeStruct((m, half), jnp.bfloat16),
        in_specs=[pl.BlockSpec(memory_space=pltpu.MemorySpace.HBM)],
        out_specs=pl.BlockSpec(memory_space=pltpu.VMEM),
        scratch_shapes=[
            pltpu.VMEM((C, mc, half), jnp.float32),
            pltpu.VMEM((m, half), jnp.float32),
            pltpu.VMEM((C, mc, half), jnp.bfloat16),
            pltpu.VMEM((C, mc, half), jnp.bfloat16),
            pltpu.SemaphoreType.DMA((C + 1,)),
            pltpu.SemaphoreType.DMA((C,)),
            pltpu.SemaphoreType.DMA((C,)),
        ],
        compiler_params=pltpu.CompilerParams(collective_id=0),
    )(x)


# device time: 6614 ns/iter; 1.0210x vs baseline; 1.0014x over previous
import jax
import jax.numpy as jnp
from jax import lax
from jax.experimental import pallas as pl
from jax.experimental.pallas import tpu as pltpu

C = 4


def kernel(x):
    _, m, n = x.shape
    half = n // 2
    mc = m // C
    x = pltpu.with_memory_space_constraint(x, pltpu.MemorySpace.HBM)

    def body(
        x_hbm,
        out_ref,
        peer_f32,
        mine_f32,
        send_buf,
        recv_buf,
        in_sems,
        send_sems,
        recv_sems,
    ):
        my_x = lax.axis_index("x")
        my_y = lax.axis_index("y")
        peer_y = 1 - my_y

        barrier_sem = pltpu.get_barrier_semaphore()
        pl.semaphore_signal(
            barrier_sem,
            inc=1,
            device_id=(my_x, peer_y),
            device_id_type=pl.DeviceIdType.MESH,
        )

        peer_dmas = []
        for c in range(C):
            d = pltpu.make_async_copy(
                x_hbm.at[0, pl.ds(c * mc, mc), pl.ds(peer_y * half, half)],
                peer_f32.at[c],
                in_sems.at[c],
            )
            d.start()
            peer_dmas.append(d)
        dma_mine = pltpu.make_async_copy(
            x_hbm.at[0, :, pl.ds(my_y * half, half)], mine_f32, in_sems.at[C]
        )
        dma_mine.start()

        rdmas = []
        for c in range(C):
            peer_dmas[c].wait()
            send_buf[c, :, :] = peer_f32[c, :, :].astype(jnp.bfloat16)
            if c == 0:
                pl.semaphore_wait(barrier_sem, 1)
            rdma = pltpu.make_async_remote_copy(
                src_ref=send_buf.at[c],
                dst_ref=recv_buf.at[c],
                send_sem=send_sems.at[c],
                recv_sem=recv_sems.at[c],
                device_id=(my_x, peer_y),
                device_id_type=pl.DeviceIdType.MESH,
            )
            rdma.start()
            rdmas.append(rdma)

        dma_mine.wait()

        for c in range(C):
            rdmas[c].wait_recv()
            out_ref[pl.ds(c * mc, mc), :] = (
                mine_f32[pl.ds(c * mc, mc), :]
                + recv_buf[c, :, :].astype(jnp.float32)
            ).astype(jnp.bfloat16)

        for c in range(C):
            rdmas[c].wait_send()

    return pl.pallas_call(
        body,
        out_shape=jax.ShapeDtypeStruct((m, half), jnp.bfloat16),
        in_specs=[pl.BlockSpec(memory_space=pltpu.MemorySpace.HBM)],
        out_specs=pl.BlockSpec(memory_space=pltpu.VMEM),
        scratch_shapes=[
            pltpu.VMEM((C, mc, half), jnp.float32),
            pltpu.VMEM((m, half), jnp.float32),
            pltpu.VMEM((C, mc, half), jnp.bfloat16),
            pltpu.VMEM((C, mc, half), jnp.bfloat16),
            pltpu.SemaphoreType.DMA((C + 1,)),
            pltpu.SemaphoreType.DMA((C,)),
            pltpu.SemaphoreType.DMA((C,)),
        ],
        compiler_params=pltpu.CompilerParams(collective_id=0),
    )(x)
